# baseline (speedup 1.0000x reference)
import functools

import jax
import jax.numpy as jnp
import numpy as np
from jax.experimental import pallas as pl
from jax.experimental.pallas import tpu as pltpu


_SLOPE = 0.1


def _phase_taps(p, q, wb):
    fine = []
    for dh in (-1, 0, 1):
        for dw in (-1, 0, 1):
            r, s = (p + dh) % 2, (q + dw) % 2
            da, db = (p + dh) // 2, (q + dw) // 2
            if db == -1:
                kind, plane = "LvarL", r
            elif db == 1:
                kind, plane = "LvarR", r
            else:
                kind, plane = "L", 2 * r + s
            fine.append((kind, plane, da, db, dh, dw))
    coarse = {}
    for dh in (-1, 0, 1):
        for dw in (-1, 0, 1):
            da, db = (p + dh) // 2, (q + dw) // 2
            coarse.setdefault((da, db), []).append((dh, dw))
    out = []
    for (da, db), taps in sorted(coarse.items()):
        kind = "OvarL" if db == -1 else ("OvarR" if db == 1 else "O")
        out.append((kind, da, db, taps))
    return fine, out


def _neck_kernel(x1p_ref, x2_ref, x3_ref,
                 w1_ref, s1_ref, b1_ref,
                 w2_ref, s2_ref, b2_ref,
                 w3_ref, s3_ref, b3_ref,
                 wf1_ref, wc1_ref, sm1_ref, bm1_ref,
                 wm2_ref, sm2_ref, bm2_ref,
                 u32_ref, mneg_ref, mpos_ref,
                 o1_ref, o2_ref, o3_ref,
                 lpad_ref, lvl_ref, lvr_ref,
                 opad_ref, ovl_ref, ovr_ref, xpad2_ref,
                 *, W2, pad):
    HW = W2 * W2

    def bn_lrelu(y, s_ref, b_ref):
        y = y * s_ref[...] + b_ref[...]
        return jnp.where(y >= 0.0, y, _SLOPE * y)

    mneg = mneg_ref[...]
    mpos = mpos_ref[...]

    lpad_ref[...] = jnp.zeros_like(lpad_ref)
    lvl_ref[...] = jnp.zeros_like(lvl_ref)
    lvr_ref[...] = jnp.zeros_like(lvr_ref)
    for i in range(4):
        li = bn_lrelu(jnp.dot(w1_ref[...], x1p_ref[0, i],
                              preferred_element_type=jnp.float32),
                      s1_ref, b1_ref).astype(jnp.bfloat16)
        lpad_ref[i, :, pad:pad + HW] = li
        r, s = i // 2, i % 2
        if s == 1:
            lvl_ref[r, :, pad:pad + HW] = li * mpos
        else:
            lvr_ref[r, :, pad:pad + HW] = li * mneg

    o2 = bn_lrelu(jnp.dot(w2_ref[...], x2_ref[0],
                          preferred_element_type=jnp.float32),
                  s2_ref, b2_ref)
    o3 = bn_lrelu(jnp.dot(w3_ref[...], x3_ref[0],
                          preferred_element_type=jnp.float32),
                  s3_ref, b3_ref)
    o3_ref[0] = o3.astype(jnp.bfloat16)

    up = jnp.dot(o3.astype(jnp.bfloat16), u32_ref[...],
                 preferred_element_type=jnp.float32)
    m2 = (o2 + up).astype(jnp.bfloat16)
    xpad2_ref[...] = jnp.zeros_like(xpad2_ref)
    xpad2_ref[:, pad:pad + HW] = m2
    patches = []
    for dh in (-1, 0, 1):
        for dw in (-1, 0, 1):
            off = dh * W2 + dw
            shifted = xpad2_ref[:, pad + off:pad + off + HW]
            if dw == -1:
                shifted = shifted * mneg
            elif dw == 1:
                shifted = shifted * mpos
            patches.append(shifted)
    p2 = jnp.concatenate(patches, axis=0)
    o2m = bn_lrelu(jnp.dot(wm2_ref[...], p2,
                           preferred_element_type=jnp.float32),
                   sm2_ref, bm2_ref)
    o2m_bf = o2m.astype(jnp.bfloat16)
    o2_ref[0] = o2m_bf

    opad_ref[...] = jnp.zeros_like(opad_ref)
    ovl_ref[...] = jnp.zeros_like(ovl_ref)
    ovr_ref[...] = jnp.zeros_like(ovr_ref)
    opad_ref[:, pad:pad + HW] = o2m_bf
    ovl_ref[:, pad:pad + HW] = o2m_bf * mpos
    ovr_ref[:, pad:pad + HW] = o2m_bf * mneg

    for ph in range(4):
        p, q = ph // 2, ph % 2
        fine, coarse = _phase_taps(p, q, W2)
        y = None
        for t, (kind, plane, da, db, _, _) in enumerate(fine):
            off = pad + da * W2 + db
            if kind == "L":
                src = lpad_ref[plane, :, off:off + HW]
            elif kind == "LvarL":
                src = lvl_ref[plane, :, off:off + HW]
            else:
                src = lvr_ref[plane, :, off:off + HW]
            d = jnp.dot(wf1_ref[ph, t], src,
                        preferred_element_type=jnp.float32)
            y = d if y is None else y + d
        for t, (kind, da, db, _) in enumerate(coarse):
            off = pad + da * W2 + db
            if kind == "O":
                src = opad_ref[:, off:off + HW]
            elif kind == "OvarL":
                src = ovl_ref[:, off:off + HW]
            else:
                src = ovr_ref[:, off:off + HW]
            y = y + jnp.dot(wc1_ref[ph, t], src,
                            preferred_element_type=jnp.float32)
        o1_ref[0, ph] = bn_lrelu(y, sm1_ref, bm1_ref).astype(jnp.bfloat16)


def _upsample_matrix(hc, wc, h, w):
    hi = (np.arange(h) * hc) // h
    wi = (np.arange(w) * wc) // w
    src = (hi[:, None] * wc + wi[None, :]).reshape(-1)
    return jnp.asarray(
        (np.arange(hc * wc)[:, None] == src[None, :]).astype(np.float32),
        dtype=jnp.bfloat16)


def kernel(x1, x2, x3,
           output1_w, output1_scale, output1_bias,
           output2_w, output2_scale, output2_bias,
           output3_w, output3_scale, output3_bias,
           merge1_w, merge1_scale, merge1_bias,
           merge2_w, merge2_scale, merge2_bias):
    N, C1, H1, W1 = x1.shape
    _, C2, H2, W2 = x2.shape
    _, C3, H3, W3 = x3.shape
    HW2, HW3 = H2 * W2, H3 * W3
    Cout = output1_w.shape[1]
    HB = H2 * W2

    x1p = (x1.reshape(N, C1, H2, 2, W2, 2)
           .transpose(0, 3, 5, 1, 2, 4)
           .reshape(N, 4, C1, HB).astype(jnp.bfloat16))
    x2f = x2.reshape(N, C2, HW2).astype(jnp.bfloat16)
    x3f = x3.reshape(N, C3, HW3).astype(jnp.bfloat16)

    def pack1x1(w):
        return w.T.astype(jnp.bfloat16)

    w1 = pack1x1(output1_w)
    w2 = pack1x1(output2_w)
    w3 = pack1x1(output3_w)
    wm2 = (jnp.transpose(merge2_w, (3, 0, 1, 2)).reshape(Cout, -1)
           .astype(jnp.bfloat16))

    wf_list, wc_list = [], []
    for ph in range(4):
        p, q = ph // 2, ph % 2
        fine, coarse = _phase_taps(p, q, W2)
        wf_list.append(jnp.stack(
            [merge1_w[dh + 1, dw + 1].T for (_, _, _, _, dh, dw) in fine]))
        wc_list.append(jnp.stack(
            [sum(merge1_w[dh + 1, dw + 1] for (dh, dw) in taps).T
             for (_, _, _, taps) in coarse]))
    wf1 = jnp.stack(wf_list).astype(jnp.bfloat16)
    wc1 = jnp.stack(wc_list).astype(jnp.bfloat16)

    u32 = _upsample_matrix(H3, W3, H2, W2)

    ww = np.arange(HB) % W2
    mneg = jnp.asarray((ww > 0).astype(np.float32).reshape(1, -1),
                       dtype=jnp.bfloat16)
    mpos = jnp.asarray((ww < W2 - 1).astype(np.float32).reshape(1, -1),
                       dtype=jnp.bfloat16)

    pad = 128
    HP = HB + 2 * pad

    const = lambda *shape: pl.BlockSpec(shape, lambda n: tuple(0 for _ in shape))

    o1f, o2f, o3f = pl.pallas_call(
        functools.partial(_neck_kernel, W2=W2, pad=pad),
        out_shape=[jax.ShapeDtypeStruct((N, 4, Cout, HB), jnp.bfloat16),
                   jax.ShapeDtypeStruct((N, Cout, HW2), jnp.bfloat16),
                   jax.ShapeDtypeStruct((N, Cout, HW3), jnp.bfloat16)],
        grid=(N,),
        in_specs=[
            pl.BlockSpec((1, 4, C1, HB), lambda n: (n, 0, 0, 0)),
            pl.BlockSpec((1, C2, HW2), lambda n: (n, 0, 0)),
            pl.BlockSpec((1, C3, HW3), lambda n: (n, 0, 0)),
            const(Cout, C1), const(Cout, 1), const(Cout, 1),
            const(Cout, C2), const(Cout, 1), const(Cout, 1),
            const(Cout, C3), const(Cout, 1), const(Cout, 1),
            const(4, 9, Cout, Cout), const(4, 4, Cout, Cout),
            const(Cout, 1), const(Cout, 1),
            const(Cout, 9 * Cout), const(Cout, 1), const(Cout, 1),
            const(HW3, HW2), const(1, HB), const(1, HB),
        ],
        out_specs=[pl.BlockSpec((1, 4, Cout, HB), lambda n: (n, 0, 0, 0)),
                   pl.BlockSpec((1, Cout, HW2), lambda n: (n, 0, 0)),
                   pl.BlockSpec((1, Cout, HW3), lambda n: (n, 0, 0))],
        scratch_shapes=[
            pltpu.VMEM((4, Cout, HP), jnp.bfloat16),
            pltpu.VMEM((2, Cout, HP), jnp.bfloat16),
            pltpu.VMEM((2, Cout, HP), jnp.bfloat16),
            pltpu.VMEM((Cout, HP), jnp.bfloat16),
            pltpu.VMEM((Cout, HP), jnp.bfloat16),
            pltpu.VMEM((Cout, HP), jnp.bfloat16),
            pltpu.VMEM((Cout, HP), jnp.bfloat16),
        ],
        compiler_params=pltpu.CompilerParams(
            dimension_semantics=("parallel",),
            vmem_limit_bytes=100 * 1024 * 1024,
        ),
    )(x1p, x2f, x3f,
      w1, output1_scale.reshape(-1, 1), output1_bias.reshape(-1, 1),
      w2, output2_scale.reshape(-1, 1), output2_bias.reshape(-1, 1),
      w3, output3_scale.reshape(-1, 1), output3_bias.reshape(-1, 1),
      wf1, wc1, merge1_scale.reshape(-1, 1), merge1_bias.reshape(-1, 1),
      wm2, merge2_scale.reshape(-1, 1), merge2_bias.reshape(-1, 1),
      u32, mneg, mpos)

    o1 = (o1f.astype(jnp.float32)
          .reshape(N, 2, 2, Cout, H2, W2)
          .transpose(0, 3, 4, 1, 5, 2)
          .reshape(N, Cout, H1, W1))
    return [o1,
            o2f.astype(jnp.float32).reshape(N, Cout, H2, W2),
            o3f.astype(jnp.float32).reshape(N, Cout, H3, W3)]


# speedup vs baseline: 1.1638x; 1.1638x over previous
import functools

import jax
import jax.numpy as jnp
import numpy as np
from jax.experimental import pallas as pl
from jax.experimental.pallas import tpu as pltpu


_SLOPE = 0.1

_TAPS = [(dh, dw) for dh in (-1, 0, 1) for dw in (-1, 0, 1)]


def _neck_kernel(x1_ref, x2_ref, x3_ref,
                 w1_ref, s1_ref, b1_ref,
                 w2_ref, s2_ref, b2_ref,
                 w3_ref, s3_ref, b3_ref,
                 wm1_ref, sm1_ref, bm1_ref,
                 wm2_ref, sm2_ref, bm2_ref,
                 u21_ref, u32_ref,
                 mneg1_ref, mpos1_ref, mneg2_ref, mpos2_ref,
                 o1_ref, o2_ref, o3_ref,
                 xp1_ref, xl1_ref, xr1_ref,
                 xp2_ref, xl2_ref, xr2_ref,
                 *, W1, W2, pad1, pad2):
    HW1 = x1_ref.shape[2]
    HW2 = x2_ref.shape[2]

    def bn_lrelu(y, s_ref, b_ref):
        y = y * s_ref[...] + b_ref[...]
        return jnp.where(y >= 0.0, y, _SLOPE * y)

    o1 = bn_lrelu(jnp.dot(w1_ref[...], x1_ref[0],
                          preferred_element_type=jnp.float32),
                  s1_ref, b1_ref)
    o2 = bn_lrelu(jnp.dot(w2_ref[...], x2_ref[0],
                          preferred_element_type=jnp.float32),
                  s2_ref, b2_ref)
    o3 = bn_lrelu(jnp.dot(w3_ref[...], x3_ref[0],
                          preferred_element_type=jnp.float32),
                  s3_ref, b3_ref)
    o3_ref[0] = o3

    def merge(o_fine, coarse_bf, u_ref, xp_ref, xl_ref, xr_ref,
              w_ref, s_ref, b_ref, pad, W, HW, mneg_ref, mpos_ref):
        up = jnp.dot(coarse_bf, u_ref[...], preferred_element_type=jnp.float32)
        m = (o_fine + up).astype(jnp.bfloat16)

        xp_ref[...] = jnp.zeros_like(xp_ref)
        xl_ref[...] = jnp.zeros_like(xl_ref)
        xr_ref[...] = jnp.zeros_like(xr_ref)
        xp_ref[:, pad:pad + HW] = m
        xl_ref[:, pad:pad + HW] = m * mpos_ref[...]
        xr_ref[:, pad:pad + HW] = m * mneg_ref[...]

        y = None
        for t, (dh, dw) in enumerate(_TAPS):
            src = xl_ref if dw == -1 else (xr_ref if dw == 1 else xp_ref)
            off = pad + dh * W + dw
            d = jnp.dot(w_ref[t], src[:, off:off + HW],
                        preferred_element_type=jnp.float32)
            y = d if y is None else y + d
        return bn_lrelu(y, s_ref, b_ref)

    o2m = merge(o2, o3.astype(jnp.bfloat16), u32_ref,
                xp2_ref, xl2_ref, xr2_ref,
                wm2_ref, sm2_ref, bm2_ref, pad2, W2, HW2,
                mneg2_ref, mpos2_ref)
    o2_ref[0] = o2m
    o1m = merge(o1, o2m.astype(jnp.bfloat16), u21_ref,
                xp1_ref, xl1_ref, xr1_ref,
                wm1_ref, sm1_ref, bm1_ref, pad1, W1, HW1,
                mneg1_ref, mpos1_ref)
    o1_ref[0] = o1m


def _upsample_matrix(hc, wc, h, w):
    hi = (np.arange(h) * hc) // h
    wi = (np.arange(w) * wc) // w
    src = (hi[:, None] * wc + wi[None, :]).reshape(-1)
    return jnp.asarray(
        (np.arange(hc * wc)[:, None] == src[None, :]).astype(np.float32),
        dtype=jnp.bfloat16)


def _col_masks(h, w):
    ww = np.arange(h * w) % w
    mneg = jnp.asarray((ww > 0).astype(np.float32).reshape(1, -1),
                       dtype=jnp.bfloat16)
    mpos = jnp.asarray((ww < w - 1).astype(np.float32).reshape(1, -1),
                       dtype=jnp.bfloat16)
    return mneg, mpos


def kernel(x1, x2, x3,
           output1_w, output1_scale, output1_bias,
           output2_w, output2_scale, output2_bias,
           output3_w, output3_scale, output3_bias,
           merge1_w, merge1_scale, merge1_bias,
           merge2_w, merge2_scale, merge2_bias):
    N, C1, H1, W1 = x1.shape
    _, C2, H2, W2 = x2.shape
    _, C3, H3, W3 = x3.shape
    HW1, HW2, HW3 = H1 * W1, H2 * W2, H3 * W3
    Cout = output1_w.shape[1]

    x1f = x1.reshape(N, C1, HW1)
    x2f = x2.reshape(N, C2, HW2)
    x3f = x3.reshape(N, C3, HW3)

    w1 = output1_w.T.astype(jnp.bfloat16)
    w2 = output2_w.T.astype(jnp.bfloat16)
    w3 = output3_w.T.astype(jnp.bfloat16)

    def pack3x3(w):
        return jnp.stack([w[dh + 1, dw + 1].T for dh, dw in _TAPS]
                         ).astype(jnp.bfloat16)

    wm1 = pack3x3(merge1_w)
    wm2 = pack3x3(merge2_w)

    u21 = _upsample_matrix(H2, W2, H1, W1)
    u32 = _upsample_matrix(H3, W3, H2, W2)
    mneg1, mpos1 = _col_masks(H1, W1)
    mneg2, mpos2 = _col_masks(H2, W2)

    pad1 = max(128, W1 + 1)
    pad2 = max(128, W2 + 1)

    const = lambda *shape: pl.BlockSpec(shape, lambda n: tuple(0 for _ in shape))
    pern = lambda c, hw: pl.BlockSpec((1, c, hw), lambda n: (n, 0, 0))

    o1f, o2f, o3f = pl.pallas_call(
        functools.partial(_neck_kernel, W1=W1, W2=W2, pad1=pad1, pad2=pad2),
        out_shape=[jax.ShapeDtypeStruct((N, Cout, HW1), jnp.float32),
                   jax.ShapeDtypeStruct((N, Cout, HW2), jnp.float32),
                   jax.ShapeDtypeStruct((N, Cout, HW3), jnp.float32)],
        grid=(N,),
        in_specs=[
            pern(C1, HW1), pern(C2, HW2), pern(C3, HW3),
            const(Cout, C1), const(Cout, 1), const(Cout, 1),
            const(Cout, C2), const(Cout, 1), const(Cout, 1),
            const(Cout, C3), const(Cout, 1), const(Cout, 1),
            const(9, Cout, Cout), const(Cout, 1), const(Cout, 1),
            const(9, Cout, Cout), const(Cout, 1), const(Cout, 1),
            const(HW2, HW1), const(HW3, HW2),
            const(1, HW1), const(1, HW1), const(1, HW2), const(1, HW2),
        ],
        out_specs=[pern(Cout, HW1), pern(Cout, HW2), pern(Cout, HW3)],
        scratch_shapes=[
            pltpu.VMEM((Cout, HW1 + 2 * pad1), jnp.bfloat16),
            pltpu.VMEM((Cout, HW1 + 2 * pad1), jnp.bfloat16),
            pltpu.VMEM((Cout, HW1 + 2 * pad1), jnp.bfloat16),
            pltpu.VMEM((Cout, HW2 + 2 * pad2), jnp.bfloat16),
            pltpu.VMEM((Cout, HW2 + 2 * pad2), jnp.bfloat16),
            pltpu.VMEM((Cout, HW2 + 2 * pad2), jnp.bfloat16),
        ],
        compiler_params=pltpu.CompilerParams(
            dimension_semantics=("parallel",),
            vmem_limit_bytes=100 * 1024 * 1024,
        ),
    )(x1f.astype(jnp.bfloat16), x2f.astype(jnp.bfloat16),
      x3f.astype(jnp.bfloat16),
      w1, output1_scale.reshape(-1, 1), output1_bias.reshape(-1, 1),
      w2, output2_scale.reshape(-1, 1), output2_bias.reshape(-1, 1),
      w3, output3_scale.reshape(-1, 1), output3_bias.reshape(-1, 1),
      wm1, merge1_scale.reshape(-1, 1), merge1_bias.reshape(-1, 1),
      wm2, merge2_scale.reshape(-1, 1), merge2_bias.reshape(-1, 1),
      u21, u32, mneg1, mpos1, mneg2, mpos2)

    return [o1f.reshape(N, Cout, H1, W1),
            o2f.reshape(N, Cout, H2, W2),
            o3f.reshape(N, Cout, H3, W3)]


# speedup vs baseline: 1.2657x; 1.0875x over previous
import functools

import jax
import jax.numpy as jnp
import numpy as np
from jax.experimental import pallas as pl
from jax.experimental.pallas import tpu as pltpu


_SLOPE = 0.1

_TAPS = [(dh, dw) for dh in (-1, 0, 1) for dw in (-1, 0, 1)]


def _neck_kernel(x1_ref, x2_ref, x3_ref,
                 w1_ref, s1_ref, b1_ref,
                 w2_ref, s2_ref, b2_ref,
                 w3_ref, s3_ref, b3_ref,
                 wm1_ref, sm1_ref, bm1_ref,
                 wm2_ref, sm2_ref, bm2_ref,
                 u21_ref, u32_ref,
                 mneg1_ref, mpos1_ref, mneg2_ref, mpos2_ref,
                 o1_ref, o2_ref, o3_ref,
                 xp1_ref, xl1_ref, xr1_ref,
                 xp2_ref, xl2_ref, xr2_ref,
                 *, W1, W2, pad1, pad2):
    HW1 = x1_ref.shape[2]
    HW2 = x2_ref.shape[2]

    def bn_lrelu(y, s_ref, b_ref):
        y = y * s_ref[...] + b_ref[...]
        return jnp.where(y >= 0.0, y, _SLOPE * y)

    o1 = bn_lrelu(jnp.dot(w1_ref[...], x1_ref[0].astype(jnp.bfloat16),
                          preferred_element_type=jnp.float32),
                  s1_ref, b1_ref)
    o2 = bn_lrelu(jnp.dot(w2_ref[...], x2_ref[0].astype(jnp.bfloat16),
                          preferred_element_type=jnp.float32),
                  s2_ref, b2_ref)
    o3 = bn_lrelu(jnp.dot(w3_ref[...], x3_ref[0].astype(jnp.bfloat16),
                          preferred_element_type=jnp.float32),
                  s3_ref, b3_ref)
    o3_ref[0] = o3

    def merge(o_fine, coarse_bf, u_ref, xp_ref, xl_ref, xr_ref,
              w_ref, s_ref, b_ref, pad, W, HW, mneg_ref, mpos_ref):
        up = jnp.dot(coarse_bf, u_ref[...], preferred_element_type=jnp.float32)
        m = (o_fine + up).astype(jnp.bfloat16)

        xp_ref[...] = jnp.zeros_like(xp_ref)
        xl_ref[...] = jnp.zeros_like(xl_ref)
        xr_ref[...] = jnp.zeros_like(xr_ref)
        xp_ref[:, pad:pad + HW] = m
        xl_ref[:, pad:pad + HW] = m * mpos_ref[...]
        xr_ref[:, pad:pad + HW] = m * mneg_ref[...]

        y = None
        for t, (dh, dw) in enumerate(_TAPS):
            src = xl_ref if dw == -1 else (xr_ref if dw == 1 else xp_ref)
            off = pad + dh * W + dw
            d = jnp.dot(w_ref[t], src[:, off:off + HW],
                        preferred_element_type=jnp.float32)
            y = d if y is None else y + d
        return bn_lrelu(y, s_ref, b_ref)

    o2m = merge(o2, o3.astype(jnp.bfloat16), u32_ref,
                xp2_ref, xl2_ref, xr2_ref,
                wm2_ref, sm2_ref, bm2_ref, pad2, W2, HW2,
                mneg2_ref, mpos2_ref)
    o2_ref[0] = o2m
    o1m = merge(o1, o2m.astype(jnp.bfloat16), u21_ref,
                xp1_ref, xl1_ref, xr1_ref,
                wm1_ref, sm1_ref, bm1_ref, pad1, W1, HW1,
                mneg1_ref, mpos1_ref)
    o1_ref[0] = o1m


def _upsample_matrix(hc, wc, h, w):
    hi = (np.arange(h) * hc) // h
    wi = (np.arange(w) * wc) // w
    src = (hi[:, None] * wc + wi[None, :]).reshape(-1)
    return jnp.asarray(
        (np.arange(hc * wc)[:, None] == src[None, :]).astype(np.float32),
        dtype=jnp.bfloat16)


def _col_masks(h, w):
    ww = np.arange(h * w) % w
    mneg = jnp.asarray((ww > 0).astype(np.float32).reshape(1, -1),
                       dtype=jnp.bfloat16)
    mpos = jnp.asarray((ww < w - 1).astype(np.float32).reshape(1, -1),
                       dtype=jnp.bfloat16)
    return mneg, mpos


def kernel(x1, x2, x3,
           output1_w, output1_scale, output1_bias,
           output2_w, output2_scale, output2_bias,
           output3_w, output3_scale, output3_bias,
           merge1_w, merge1_scale, merge1_bias,
           merge2_w, merge2_scale, merge2_bias):
    N, C1, H1, W1 = x1.shape
    _, C2, H2, W2 = x2.shape
    _, C3, H3, W3 = x3.shape
    HW1, HW2, HW3 = H1 * W1, H2 * W2, H3 * W3
    Cout = output1_w.shape[1]

    x1f = x1.reshape(N, C1, HW1)
    x2f = x2.reshape(N, C2, HW2)
    x3f = x3.reshape(N, C3, HW3)

    w1 = output1_w.T.astype(jnp.bfloat16)
    w2 = output2_w.T.astype(jnp.bfloat16)
    w3 = output3_w.T.astype(jnp.bfloat16)

    def pack3x3(w):
        return jnp.stack([w[dh + 1, dw + 1].T for dh, dw in _TAPS]
                         ).astype(jnp.bfloat16)

    wm1 = pack3x3(merge1_w)
    wm2 = pack3x3(merge2_w)

    u21 = _upsample_matrix(H2, W2, H1, W1)
    u32 = _upsample_matrix(H3, W3, H2, W2)
    mneg1, mpos1 = _col_masks(H1, W1)
    mneg2, mpos2 = _col_masks(H2, W2)

    pad1 = max(128, W1 + 1)
    pad2 = max(128, W2 + 1)

    const = lambda *shape: pl.BlockSpec(shape, lambda n: tuple(0 for _ in shape))
    pern = lambda c, hw: pl.BlockSpec((1, c, hw), lambda n: (n, 0, 0))

    o1f, o2f, o3f = pl.pallas_call(
        functools.partial(_neck_kernel, W1=W1, W2=W2, pad1=pad1, pad2=pad2),
        out_shape=[jax.ShapeDtypeStruct((N, Cout, HW1), jnp.float32),
                   jax.ShapeDtypeStruct((N, Cout, HW2), jnp.float32),
                   jax.ShapeDtypeStruct((N, Cout, HW3), jnp.float32)],
        grid=(N,),
        in_specs=[
            pern(C1, HW1), pern(C2, HW2), pern(C3, HW3),
            const(Cout, C1), const(Cout, 1), const(Cout, 1),
            const(Cout, C2), const(Cout, 1), const(Cout, 1),
            const(Cout, C3), const(Cout, 1), const(Cout, 1),
            const(9, Cout, Cout), const(Cout, 1), const(Cout, 1),
            const(9, Cout, Cout), const(Cout, 1), const(Cout, 1),
            const(HW2, HW1), const(HW3, HW2),
            const(1, HW1), const(1, HW1), const(1, HW2), const(1, HW2),
        ],
        out_specs=[pern(Cout, HW1), pern(Cout, HW2), pern(Cout, HW3)],
        scratch_shapes=[
            pltpu.VMEM((Cout, HW1 + 2 * pad1), jnp.bfloat16),
            pltpu.VMEM((Cout, HW1 + 2 * pad1), jnp.bfloat16),
            pltpu.VMEM((Cout, HW1 + 2 * pad1), jnp.bfloat16),
            pltpu.VMEM((Cout, HW2 + 2 * pad2), jnp.bfloat16),
            pltpu.VMEM((Cout, HW2 + 2 * pad2), jnp.bfloat16),
            pltpu.VMEM((Cout, HW2 + 2 * pad2), jnp.bfloat16),
        ],
        compiler_params=pltpu.CompilerParams(
            dimension_semantics=("parallel",),
            vmem_limit_bytes=100 * 1024 * 1024,
        ),
    )(x1f, x2f, x3f,
      w1, output1_scale.reshape(-1, 1), output1_bias.reshape(-1, 1),
      w2, output2_scale.reshape(-1, 1), output2_bias.reshape(-1, 1),
      w3, output3_scale.reshape(-1, 1), output3_bias.reshape(-1, 1),
      wm1, merge1_scale.reshape(-1, 1), merge1_bias.reshape(-1, 1),
      wm2, merge2_scale.reshape(-1, 1), merge2_bias.reshape(-1, 1),
      u21, u32, mneg1, mpos1, mneg2, mpos2)

    return [o1f.reshape(N, Cout, H1, W1),
            o2f.reshape(N, Cout, H2, W2),
            o3f.reshape(N, Cout, H3, W3)]


# speedup vs baseline: 1.3034x; 1.0298x over previous
import functools

import jax
import jax.numpy as jnp
import numpy as np
from jax.experimental import pallas as pl
from jax.experimental.pallas import tpu as pltpu


_SLOPE = 0.1

_TAPS = [(dh, dw) for dh in (-1, 0, 1) for dw in (-1, 0, 1)]


def _neck_kernel(x1_ref, x2_ref, x3_ref,
                 w1_ref, b1_ref,
                 w2_ref, b2_ref,
                 w3_ref, b3_ref,
                 wm1_ref, bm1_ref,
                 wm2_ref, bm2_ref,
                 u21_ref, u32_ref,
                 mneg1_ref, mpos1_ref, mneg2_ref, mpos2_ref,
                 o1_ref, o2_ref, o3_ref,
                 xp1_ref, xl1_ref, xr1_ref,
                 xp2_ref, xl2_ref, xr2_ref,
                 *, W1, W2, pad1, pad2):
    HW1 = x1_ref.shape[2]
    HW2 = x2_ref.shape[2]

    def bias_lrelu(y, b_ref):
        y = y + b_ref[...]
        return jnp.where(y >= 0.0, y, _SLOPE * y)

    def lateral(w_ref, x_ref, b_ref):
        y = jnp.dot(w_ref[...], x_ref[0].astype(jnp.bfloat16),
                    preferred_element_type=jnp.float32)
        return bias_lrelu(y, b_ref)

    def merge(fine_bf, coarse_bf, u_ref, xp_ref, xl_ref, xr_ref,
              w_ref, b_ref, pad, W, HW, mneg_ref, mpos_ref):
        up = jnp.dot(coarse_bf, u_ref[...],
                     preferred_element_type=jnp.float32)
        m = fine_bf + up.astype(jnp.bfloat16)

        for ref in (xp_ref, xl_ref, xr_ref):
            ref[:, :pad] = jnp.zeros_like(ref[:, :pad])
            ref[:, pad + HW:] = jnp.zeros_like(ref[:, pad + HW:])
        xp_ref[:, pad:pad + HW] = m
        xl_ref[:, pad:pad + HW] = m * mpos_ref[...]
        xr_ref[:, pad:pad + HW] = m * mneg_ref[...]

        y = None
        for t, (dh, dw) in enumerate(_TAPS):
            src = xl_ref if dw == -1 else (xr_ref if dw == 1 else xp_ref)
            off = pad + dh * W + dw
            d = jnp.dot(w_ref[t], src[:, off:off + HW],
                        preferred_element_type=jnp.float32)
            y = d if y is None else y + d
        return bias_lrelu(y, b_ref)

    o3 = lateral(w3_ref, x3_ref, b3_ref)
    o3_ref[0] = o3
    o2 = lateral(w2_ref, x2_ref, b2_ref)
    o2m = merge(o2.astype(jnp.bfloat16), o3.astype(jnp.bfloat16), u32_ref,
                xp2_ref, xl2_ref, xr2_ref,
                wm2_ref, bm2_ref, pad2, W2, HW2, mneg2_ref, mpos2_ref)
    o2_ref[0] = o2m
    o1 = lateral(w1_ref, x1_ref, b1_ref)
    o1m = merge(o1.astype(jnp.bfloat16), o2m.astype(jnp.bfloat16), u21_ref,
                xp1_ref, xl1_ref, xr1_ref,
                wm1_ref, bm1_ref, pad1, W1, HW1, mneg1_ref, mpos1_ref)
    o1_ref[0] = o1m


def _upsample_matrix(hc, wc, h, w):
    hi = (np.arange(h) * hc) // h
    wi = (np.arange(w) * wc) // w
    src = (hi[:, None] * wc + wi[None, :]).reshape(-1)
    return jnp.asarray(
        (np.arange(hc * wc)[:, None] == src[None, :]).astype(np.float32),
        dtype=jnp.bfloat16)


def _col_masks(h, w):
    ww = np.arange(h * w) % w
    mneg = jnp.asarray((ww > 0).astype(np.float32).reshape(1, -1),
                       dtype=jnp.bfloat16)
    mpos = jnp.asarray((ww < w - 1).astype(np.float32).reshape(1, -1),
                       dtype=jnp.bfloat16)
    return mneg, mpos


def kernel(x1, x2, x3,
           output1_w, output1_scale, output1_bias,
           output2_w, output2_scale, output2_bias,
           output3_w, output3_scale, output3_bias,
           merge1_w, merge1_scale, merge1_bias,
           merge2_w, merge2_scale, merge2_bias):
    N, C1, H1, W1 = x1.shape
    _, C2, H2, W2 = x2.shape
    _, C3, H3, W3 = x3.shape
    HW1, HW2, HW3 = H1 * W1, H2 * W2, H3 * W3
    Cout = output1_w.shape[1]

    x1f = x1.reshape(N, C1, HW1)
    x2f = x2.reshape(N, C2, HW2)
    x3f = x3.reshape(N, C3, HW3)

    def pack1x1(w, scale):
        return (scale[:, None] * w.T).astype(jnp.bfloat16)

    w1 = pack1x1(output1_w, output1_scale)
    w2 = pack1x1(output2_w, output2_scale)
    w3 = pack1x1(output3_w, output3_scale)

    def pack3x3(w, scale):
        return jnp.stack([scale[:, None] * w[dh + 1, dw + 1].T
                          for dh, dw in _TAPS]).astype(jnp.bfloat16)

    wm1 = pack3x3(merge1_w, merge1_scale)
    wm2 = pack3x3(merge2_w, merge2_scale)

    u21 = _upsample_matrix(H2, W2, H1, W1)
    u32 = _upsample_matrix(H3, W3, H2, W2)
    mneg1, mpos1 = _col_masks(H1, W1)
    mneg2, mpos2 = _col_masks(H2, W2)

    pad1 = max(128, W1 + 1)
    pad2 = max(128, W2 + 1)

    const = lambda *shape: pl.BlockSpec(shape, lambda n: tuple(0 for _ in shape))
    pern = lambda c, hw: pl.BlockSpec((1, c, hw), lambda n: (n, 0, 0))

    o1f, o2f, o3f = pl.pallas_call(
        functools.partial(_neck_kernel, W1=W1, W2=W2, pad1=pad1, pad2=pad2),
        out_shape=[jax.ShapeDtypeStruct((N, Cout, HW1), jnp.float32),
                   jax.ShapeDtypeStruct((N, Cout, HW2), jnp.float32),
                   jax.ShapeDtypeStruct((N, Cout, HW3), jnp.float32)],
        grid=(N,),
        in_specs=[
            pern(C1, HW1), pern(C2, HW2), pern(C3, HW3),
            const(Cout, C1), const(Cout, 1),
            const(Cout, C2), const(Cout, 1),
            const(Cout, C3), const(Cout, 1),
            const(9, Cout, Cout), const(Cout, 1),
            const(9, Cout, Cout), const(Cout, 1),
            const(HW2, HW1), const(HW3, HW2),
            const(1, HW1), const(1, HW1), const(1, HW2), const(1, HW2),
        ],
        out_specs=[pern(Cout, HW1), pern(Cout, HW2), pern(Cout, HW3)],
        scratch_shapes=[
            pltpu.VMEM((Cout, HW1 + 2 * pad1), jnp.bfloat16),
            pltpu.VMEM((Cout, HW1 + 2 * pad1), jnp.bfloat16),
            pltpu.VMEM((Cout, HW1 + 2 * pad1), jnp.bfloat16),
            pltpu.VMEM((Cout, HW2 + 2 * pad2), jnp.bfloat16),
            pltpu.VMEM((Cout, HW2 + 2 * pad2), jnp.bfloat16),
            pltpu.VMEM((Cout, HW2 + 2 * pad2), jnp.bfloat16),
        ],
        compiler_params=pltpu.CompilerParams(
            dimension_semantics=("parallel",),
            vmem_limit_bytes=100 * 1024 * 1024,
        ),
    )(x1f, x2f, x3f,
      w1, output1_bias.reshape(-1, 1),
      w2, output2_bias.reshape(-1, 1),
      w3, output3_bias.reshape(-1, 1),
      wm1, merge1_bias.reshape(-1, 1),
      wm2, merge2_bias.reshape(-1, 1),
      u21, u32, mneg1, mpos1, mneg2, mpos2)

    return [o1f.reshape(N, Cout, H1, W1),
            o2f.reshape(N, Cout, H2, W2),
            o3f.reshape(N, Cout, H3, W3)]
